# baseline (speedup 1.0000x reference)
import functools

import jax
import jax.numpy as jnp
from jax import lax
from jax.experimental import pallas as pl
from jax.experimental.pallas import tpu as pltpu
from jax.experimental.pallas import tpu_sc as plsc

N = 50000
E = 800000
F_IN = 4
H = 64

NP = 50176
EP = 819200
NC = 2
NS = 16
NW = NC * NS
PT = EP // NW
GT = PT * F_IN
CH = 6400
ZS = NP // NS

_mesh = plsc.VectorSubcoreMesh(core_axis_name="c", subcore_axis_name="s")


def _zero_vmem(buf, n):
    z = jnp.zeros((16,), jnp.float32)

    @plsc.parallel_loop(0, n // 16, unroll=8)
    def _(i):
        buf[pl.ds(pl.multiple_of(i * 16, 16), 16)] = z


@functools.partial(
    pl.kernel,
    out_type=jax.ShapeDtypeStruct((NW * NP,), jnp.float32),
    mesh=_mesh,
    compiler_params=pltpu.CompilerParams(needs_layout_passes=False),
    scratch_types=[
        pltpu.VMEM((NP,), jnp.float32),
        pltpu.VMEM((CH,), jnp.int32),
        pltpu.VMEM((CH,), jnp.float32),
    ],
)
def _deg_sc(dst_hbm, w_hbm, out_hbm, acc, di, wv):
    c = lax.axis_index("c")
    s = lax.axis_index("s")
    _zero_vmem(acc, NP)
    base = c * (EP // NC) + s * PT
    for k in range(PT // CH):
        off = base + k * CH
        pltpu.sync_copy(dst_hbm.at[pl.ds(off, CH)], di)
        pltpu.sync_copy(w_hbm.at[pl.ds(off, CH)], wv)

        def body(i, carry):
            for u in range(8):
                d16 = pl.ds(pl.multiple_of(i * 128 + u * 16, 16), 16)
                plsc.addupdate_scatter(acc, [di[d16]], wv[d16])
            return carry

        lax.fori_loop(0, CH // 128, body, 0)
    pltpu.sync_copy(acc, out_hbm.at[pl.ds((c * NS + s) * NP, NP)])


@functools.partial(
    pl.kernel,
    out_type=jax.ShapeDtypeStruct((F_IN * NC * (NS // F_IN) * NP,),
                                  jnp.float32),
    mesh=_mesh,
    compiler_params=pltpu.CompilerParams(needs_layout_passes=False),
    scratch_types=[
        pltpu.VMEM((NP,), jnp.float32),
        pltpu.VMEM((NP,), jnp.float32),
        pltpu.VMEM((CH,), jnp.int32),
        pltpu.VMEM((CH,), jnp.int32),
        pltpu.VMEM((CH,), jnp.float32),
    ],
)
def _agg_sc(src_hbm, dst_hbm, w_hbm, x0, x1, x2, x3, dis_hbm, out_hbm,
            col, acc, si, di, wv):
    c = lax.axis_index("c")
    s = lax.axis_index("s")
    f = s % F_IN
    g = s // F_IN
    xf = (x0, x1, x2, x3)

    pltpu.sync_copy(dis_hbm.at[pl.ds(0, NP)], acc)
    for ff in range(F_IN):

        @pl.when(f == ff)
        def _():
            pltpu.sync_copy(xf[ff].at[pl.ds(0, NP)], col)

    @plsc.parallel_loop(0, NP // 16, unroll=8)
    def _(i):
        d16 = pl.ds(pl.multiple_of(i * 16, 16), 16)
        col[d16] = col[d16] * acc[d16]
    _zero_vmem(acc, NP)

    base = c * (EP // NC) + g * GT
    for k in range(GT // CH):
        off = base + k * CH
        pltpu.sync_copy(src_hbm.at[pl.ds(off, CH)], si)
        pltpu.sync_copy(dst_hbm.at[pl.ds(off, CH)], di)
        pltpu.sync_copy(w_hbm.at[pl.ds(off, CH)], wv)

        def body(i, carry):
            for u in range(8):
                d16 = pl.ds(pl.multiple_of(i * 128 + u * 16, 16), 16)
                v = plsc.load_gather(col, [si[d16]])
                plsc.addupdate_scatter(acc, [di[d16]], v * wv[d16])
            return carry

        lax.fori_loop(0, CH // 128, body, 0)
    slot = f * (NC * (NS // F_IN)) + c * (NS // F_IN) + g
    pltpu.sync_copy(acc, out_hbm.at[pl.ds(slot * NP, NP)])


@functools.partial(
    pl.kernel,
    out_type=jax.ShapeDtypeStruct((NW * NP,), jnp.float32),
    mesh=_mesh,
    compiler_params=pltpu.CompilerParams(needs_layout_passes=False),
    scratch_types=[
        pltpu.VMEM((NP,), jnp.float32),
        pltpu.VMEM((NP,), jnp.float32),
        pltpu.VMEM((CH,), jnp.int32),
        pltpu.VMEM((CH,), jnp.int32),
        pltpu.VMEM((CH,), jnp.float32),
    ],
)
def _out_sc(src_hbm, dst_hbm, w_hbm, ys_hbm, out_hbm, col, acc, si, di, wv):
    c = lax.axis_index("c")
    s = lax.axis_index("s")
    pltpu.sync_copy(ys_hbm.at[pl.ds(0, NP)], col)
    _zero_vmem(acc, NP)
    base = c * (EP // NC) + s * PT
    for k in range(PT // CH):
        off = base + k * CH
        pltpu.sync_copy(src_hbm.at[pl.ds(off, CH)], si)
        pltpu.sync_copy(dst_hbm.at[pl.ds(off, CH)], di)
        pltpu.sync_copy(w_hbm.at[pl.ds(off, CH)], wv)

        def body(i, carry):
            for u in range(8):
                d16 = pl.ds(pl.multiple_of(i * 128 + u * 16, 16), 16)
                v = plsc.load_gather(col, [si[d16]])
                plsc.addupdate_scatter(acc, [di[d16]], v * wv[d16])
            return carry

        lax.fori_loop(0, CH // 128, body, 0)
    pltpu.sync_copy(acc, out_hbm.at[pl.ds((c * NS + s) * NP, NP)])


def _p2_body(degp_ref, dis_ref, selfn_ref):
    deg = jnp.sum(degp_ref[...], axis=0, keepdims=True) + 1.0
    dis = lax.rsqrt(deg)
    dis_ref[...] = dis
    selfn_ref[...] = dis * dis


_p2_tc = pl.pallas_call(
    _p2_body,
    out_shape=(
        jax.ShapeDtypeStruct((1, NP), jnp.float32),
        jax.ShapeDtypeStruct((1, NP), jnp.float32),
    ),
)


_B4 = NP // 8
_G3 = NC * (NS // F_IN)


def _p4_body(ag0, ag1, ag2, ag3, xt_ref, dis_ref, selfn_ref,
             w1t_ref, b1c_ref, w2_ref, b2s_ref, ys_ref, oinit_ref):
    dis = dis_ref[...]
    selfn = selfn_ref[...]
    xt = xt_ref[...]
    aggs = []
    for f, ag in enumerate((ag0, ag1, ag2, ag3)):
        tot = jnp.sum(ag[...], axis=0, keepdims=True)
        aggs.append(dis * tot + selfn * xt[f:f + 1, :])
    w1t = w1t_ref[...]
    h = b1c_ref[...]
    for f in range(F_IN):
        h = h + w1t[:, f:f + 1] * aggs[f]
    h = jnp.maximum(h, 0.0)
    y = jnp.sum(h * w2_ref[...], axis=0, keepdims=True)
    ys_ref[...] = y * dis
    oinit_ref[...] = y * selfn + b2s_ref[...]


_p4_tc = pl.pallas_call(
    _p4_body,
    grid=(NP // _B4,),
    in_specs=[
        pl.BlockSpec((_G3, _B4), lambda i: (0, i)),
        pl.BlockSpec((_G3, _B4), lambda i: (0, i)),
        pl.BlockSpec((_G3, _B4), lambda i: (0, i)),
        pl.BlockSpec((_G3, _B4), lambda i: (0, i)),
        pl.BlockSpec((F_IN, _B4), lambda i: (0, i)),
        pl.BlockSpec((1, _B4), lambda i: (0, i)),
        pl.BlockSpec((1, _B4), lambda i: (0, i)),
        pl.BlockSpec((H, F_IN), lambda i: (0, 0)),
        pl.BlockSpec((H, 1), lambda i: (0, 0)),
        pl.BlockSpec((H, 1), lambda i: (0, 0)),
        pl.BlockSpec((1, 1), lambda i: (0, 0)),
    ],
    out_specs=(
        pl.BlockSpec((1, _B4), lambda i: (0, i)),
        pl.BlockSpec((1, _B4), lambda i: (0, i)),
    ),
    out_shape=(
        jax.ShapeDtypeStruct((1, NP), jnp.float32),
        jax.ShapeDtypeStruct((1, NP), jnp.float32),
    ),
)


def _p6_body(outp_ref, dis_ref, oinit_ref, out_ref):
    tot = jnp.sum(outp_ref[...], axis=0, keepdims=True)
    out_ref[...] = oinit_ref[...] + dis_ref[...] * tot


_p6_tc = pl.pallas_call(
    _p6_body,
    out_shape=jax.ShapeDtypeStruct((1, NP), jnp.float32),
)


def kernel(x, edge_index, edge_attr, W1, b1, W2, b2):
    f32 = jnp.float32
    pad_e = EP - E
    srcp = jnp.concatenate(
        [edge_index[0], jnp.full((pad_e,), N, dtype=jnp.int32)])
    dstp = jnp.concatenate(
        [edge_index[1], jnp.full((pad_e,), N, dtype=jnp.int32)])
    wp = jnp.concatenate([edge_attr.astype(f32), jnp.zeros((pad_e,), f32)])
    xpad = jnp.pad(x.astype(f32), ((0, NP - N), (0, 0)))
    xt = xpad.T
    xcols = [xt[ff].reshape(NP) for ff in range(F_IN)]

    degp = _deg_sc(dstp, wp).reshape(NW, NP)
    dis, selfn = _p2_tc(degp)
    agg = _agg_sc(srcp, dstp, wp, xcols[0], xcols[1], xcols[2], xcols[3],
                  dis.reshape(NP)).reshape(F_IN, _G3, NP)
    ys, oinit = _p4_tc(agg[0], agg[1], agg[2], agg[3], xt, dis, selfn,
                       W1.astype(f32).T, b1.astype(f32).reshape(H, 1),
                       W2.astype(f32), b2.astype(f32).reshape(1, 1))
    outp = _out_sc(srcp, dstp, wp, ys.reshape(NP)).reshape(NW, NP)
    out = _p6_tc(outp, dis, oinit)
    return out[0, :N].reshape(N, 1)


# speedup vs baseline: 1.0005x; 1.0005x over previous
import functools

import jax
import jax.numpy as jnp
from jax import lax
from jax.experimental import pallas as pl
from jax.experimental.pallas import tpu as pltpu
from jax.experimental.pallas import tpu_sc as plsc

N = 50000
E = 800000
F_IN = 4
H = 64

NP = 50176
EP = 819200
NC = 2
NS = 16
NW = NC * NS
PT = EP // NW
GT = PT * F_IN
CH = 6400
ZS = NP // NS

_mesh = plsc.VectorSubcoreMesh(core_axis_name="c", subcore_axis_name="s")


def _zero_vmem(buf, n):
    z = jnp.zeros((16,), jnp.float32)

    @plsc.parallel_loop(0, n // 16, unroll=8)
    def _(i):
        buf[pl.ds(pl.multiple_of(i * 16, 16), 16)] = z


@functools.partial(
    pl.kernel,
    out_type=jax.ShapeDtypeStruct((NW * NP,), jnp.float32),
    mesh=_mesh,
    compiler_params=pltpu.CompilerParams(needs_layout_passes=False),
    scratch_types=[
        pltpu.VMEM((NP,), jnp.float32),
        pltpu.VMEM((CH,), jnp.int32),
        pltpu.VMEM((CH,), jnp.float32),
    ],
)
def _deg_sc(dst_hbm, w_hbm, out_hbm, acc, di, wv):
    c = lax.axis_index("c")
    s = lax.axis_index("s")
    _zero_vmem(acc, NP)
    base = c * (EP // NC) + s * PT
    for k in range(PT // CH):
        off = base + k * CH
        pltpu.sync_copy(dst_hbm.at[pl.ds(off, CH)], di)
        pltpu.sync_copy(w_hbm.at[pl.ds(off, CH)], wv)

        def body(i, carry):
            for u in range(4):
                d16 = pl.ds(pl.multiple_of(i * 64 + u * 16, 16), 16)
                plsc.addupdate_scatter(acc, [di[d16]], wv[d16])
            return carry

        lax.fori_loop(0, CH // 64, body, 0)
    pltpu.sync_copy(acc, out_hbm.at[pl.ds((c * NS + s) * NP, NP)])


@functools.partial(
    pl.kernel,
    out_type=jax.ShapeDtypeStruct((F_IN * NC * (NS // F_IN) * NP,),
                                  jnp.float32),
    mesh=_mesh,
    compiler_params=pltpu.CompilerParams(needs_layout_passes=False),
    scratch_types=[
        pltpu.VMEM((NP,), jnp.float32),
        pltpu.VMEM((NP,), jnp.float32),
        pltpu.VMEM((CH,), jnp.int32),
        pltpu.VMEM((CH,), jnp.int32),
        pltpu.VMEM((CH,), jnp.float32),
    ],
)
def _agg_sc(src_hbm, dst_hbm, w_hbm, x0, x1, x2, x3, dis_hbm, out_hbm,
            col, acc, si, di, wv):
    c = lax.axis_index("c")
    s = lax.axis_index("s")
    f = s % F_IN
    g = s // F_IN
    xf = (x0, x1, x2, x3)

    pltpu.sync_copy(dis_hbm.at[pl.ds(0, NP)], acc)
    for ff in range(F_IN):

        @pl.when(f == ff)
        def _():
            pltpu.sync_copy(xf[ff].at[pl.ds(0, NP)], col)

    @plsc.parallel_loop(0, NP // 16, unroll=8)
    def _(i):
        d16 = pl.ds(pl.multiple_of(i * 16, 16), 16)
        col[d16] = col[d16] * acc[d16]
    _zero_vmem(acc, NP)

    base = c * (EP // NC) + g * GT
    for k in range(GT // CH):
        off = base + k * CH
        pltpu.sync_copy(src_hbm.at[pl.ds(off, CH)], si)
        pltpu.sync_copy(dst_hbm.at[pl.ds(off, CH)], di)
        pltpu.sync_copy(w_hbm.at[pl.ds(off, CH)], wv)

        def body(i, carry):
            for u in range(4):
                d16 = pl.ds(pl.multiple_of(i * 64 + u * 16, 16), 16)
                v = plsc.load_gather(col, [si[d16]])
                plsc.addupdate_scatter(acc, [di[d16]], v * wv[d16])
            return carry

        lax.fori_loop(0, CH // 64, body, 0)
    slot = f * (NC * (NS // F_IN)) + c * (NS // F_IN) + g
    pltpu.sync_copy(acc, out_hbm.at[pl.ds(slot * NP, NP)])


@functools.partial(
    pl.kernel,
    out_type=jax.ShapeDtypeStruct((NW * NP,), jnp.float32),
    mesh=_mesh,
    compiler_params=pltpu.CompilerParams(needs_layout_passes=False),
    scratch_types=[
        pltpu.VMEM((NP,), jnp.float32),
        pltpu.VMEM((NP,), jnp.float32),
        pltpu.VMEM((CH,), jnp.int32),
        pltpu.VMEM((CH,), jnp.int32),
        pltpu.VMEM((CH,), jnp.float32),
    ],
)
def _out_sc(src_hbm, dst_hbm, w_hbm, ys_hbm, out_hbm, col, acc, si, di, wv):
    c = lax.axis_index("c")
    s = lax.axis_index("s")
    pltpu.sync_copy(ys_hbm.at[pl.ds(0, NP)], col)
    _zero_vmem(acc, NP)
    base = c * (EP // NC) + s * PT
    for k in range(PT // CH):
        off = base + k * CH
        pltpu.sync_copy(src_hbm.at[pl.ds(off, CH)], si)
        pltpu.sync_copy(dst_hbm.at[pl.ds(off, CH)], di)
        pltpu.sync_copy(w_hbm.at[pl.ds(off, CH)], wv)

        def body(i, carry):
            for u in range(4):
                d16 = pl.ds(pl.multiple_of(i * 64 + u * 16, 16), 16)
                v = plsc.load_gather(col, [si[d16]])
                plsc.addupdate_scatter(acc, [di[d16]], v * wv[d16])
            return carry

        lax.fori_loop(0, CH // 64, body, 0)
    pltpu.sync_copy(acc, out_hbm.at[pl.ds((c * NS + s) * NP, NP)])


def _p2_body(degp_ref, dis_ref, selfn_ref):
    deg = jnp.sum(degp_ref[...], axis=0, keepdims=True) + 1.0
    dis = lax.rsqrt(deg)
    dis_ref[...] = dis
    selfn_ref[...] = dis * dis


_p2_tc = pl.pallas_call(
    _p2_body,
    out_shape=(
        jax.ShapeDtypeStruct((1, NP), jnp.float32),
        jax.ShapeDtypeStruct((1, NP), jnp.float32),
    ),
)


_B4 = NP // 8
_G3 = NC * (NS // F_IN)


def _p4_body(ag0, ag1, ag2, ag3, xt_ref, dis_ref, selfn_ref,
             w1t_ref, b1c_ref, w2_ref, b2s_ref, ys_ref, oinit_ref):
    dis = dis_ref[...]
    selfn = selfn_ref[...]
    xt = xt_ref[...]
    aggs = []
    for f, ag in enumerate((ag0, ag1, ag2, ag3)):
        tot = jnp.sum(ag[...], axis=0, keepdims=True)
        aggs.append(dis * tot + selfn * xt[f:f + 1, :])
    w1t = w1t_ref[...]
    h = b1c_ref[...]
    for f in range(F_IN):
        h = h + w1t[:, f:f + 1] * aggs[f]
    h = jnp.maximum(h, 0.0)
    y = jnp.sum(h * w2_ref[...], axis=0, keepdims=True)
    ys_ref[...] = y * dis
    oinit_ref[...] = y * selfn + b2s_ref[...]


_p4_tc = pl.pallas_call(
    _p4_body,
    grid=(NP // _B4,),
    in_specs=[
        pl.BlockSpec((_G3, _B4), lambda i: (0, i)),
        pl.BlockSpec((_G3, _B4), lambda i: (0, i)),
        pl.BlockSpec((_G3, _B4), lambda i: (0, i)),
        pl.BlockSpec((_G3, _B4), lambda i: (0, i)),
        pl.BlockSpec((F_IN, _B4), lambda i: (0, i)),
        pl.BlockSpec((1, _B4), lambda i: (0, i)),
        pl.BlockSpec((1, _B4), lambda i: (0, i)),
        pl.BlockSpec((H, F_IN), lambda i: (0, 0)),
        pl.BlockSpec((H, 1), lambda i: (0, 0)),
        pl.BlockSpec((H, 1), lambda i: (0, 0)),
        pl.BlockSpec((1, 1), lambda i: (0, 0)),
    ],
    out_specs=(
        pl.BlockSpec((1, _B4), lambda i: (0, i)),
        pl.BlockSpec((1, _B4), lambda i: (0, i)),
    ),
    out_shape=(
        jax.ShapeDtypeStruct((1, NP), jnp.float32),
        jax.ShapeDtypeStruct((1, NP), jnp.float32),
    ),
)


def _p6_body(outp_ref, dis_ref, oinit_ref, out_ref):
    tot = jnp.sum(outp_ref[...], axis=0, keepdims=True)
    out_ref[...] = oinit_ref[...] + dis_ref[...] * tot


_p6_tc = pl.pallas_call(
    _p6_body,
    out_shape=jax.ShapeDtypeStruct((1, NP), jnp.float32),
)


def kernel(x, edge_index, edge_attr, W1, b1, W2, b2):
    f32 = jnp.float32
    pad_e = EP - E
    srcp = jnp.concatenate(
        [edge_index[0], jnp.full((pad_e,), N, dtype=jnp.int32)])
    dstp = jnp.concatenate(
        [edge_index[1], jnp.full((pad_e,), N, dtype=jnp.int32)])
    wp = jnp.concatenate([edge_attr.astype(f32), jnp.zeros((pad_e,), f32)])
    xpad = jnp.pad(x.astype(f32), ((0, NP - N), (0, 0)))
    xt = xpad.T
    xcols = [xt[ff].reshape(NP) for ff in range(F_IN)]

    degp = _deg_sc(dstp, wp).reshape(NW, NP)
    dis, selfn = _p2_tc(degp)
    agg = _agg_sc(srcp, dstp, wp, xcols[0], xcols[1], xcols[2], xcols[3],
                  dis.reshape(NP)).reshape(F_IN, _G3, NP)
    ys, oinit = _p4_tc(agg[0], agg[1], agg[2], agg[3], xt, dis, selfn,
                       W1.astype(f32).T, b1.astype(f32).reshape(H, 1),
                       W2.astype(f32), b2.astype(f32).reshape(1, 1))
    outp = _out_sc(srcp, dstp, wp, ys.reshape(NP)).reshape(NW, NP)
    out = _p6_tc(outp, dis, oinit)
    return out[0, :N].reshape(N, 1)


# speedup vs baseline: 1.1236x; 1.1230x over previous
import functools

import jax
import jax.numpy as jnp
from jax import lax
from jax.experimental import pallas as pl
from jax.experimental.pallas import tpu as pltpu
from jax.experimental.pallas import tpu_sc as plsc

N = 50000
E = 800000
F_IN = 4
H = 64

NP = 50176
EP = 819200
NC = 2
NS = 16
NW = NC * NS
PT = EP // NW
GT = PT * F_IN
CH = 6400
CHA = 3200
ZS = NP // NS

_mesh = plsc.VectorSubcoreMesh(core_axis_name="c", subcore_axis_name="s")


def _zero_vmem(buf, n):
    z = jnp.zeros((16,), jnp.float32)

    @plsc.parallel_loop(0, n // 16, unroll=8)
    def _(i):
        buf[pl.ds(pl.multiple_of(i * 16, 16), 16)] = z


@functools.partial(
    pl.kernel,
    out_type=jax.ShapeDtypeStruct((NW * NP,), jnp.float32),
    mesh=_mesh,
    compiler_params=pltpu.CompilerParams(needs_layout_passes=False),
    scratch_types=[
        pltpu.VMEM((NP,), jnp.float32),
        pltpu.VMEM((CH,), jnp.int32),
        pltpu.VMEM((CH,), jnp.float32),
    ],
)
def _deg_sc(dst_hbm, w_hbm, out_hbm, acc, di, wv):
    c = lax.axis_index("c")
    s = lax.axis_index("s")
    _zero_vmem(acc, NP)
    base = c * (EP // NC) + s * PT
    for k in range(PT // CH):
        off = base + k * CH
        pltpu.sync_copy(dst_hbm.at[pl.ds(off, CH)], di)
        pltpu.sync_copy(w_hbm.at[pl.ds(off, CH)], wv)

        def body(i, carry):
            for u in range(4):
                d16 = pl.ds(pl.multiple_of(i * 64 + u * 16, 16), 16)
                plsc.addupdate_scatter(acc, [di[d16]], wv[d16])
            return carry

        lax.fori_loop(0, CH // 64, body, 0)
    pltpu.sync_copy(acc, out_hbm.at[pl.ds((c * NS + s) * NP, NP)])


@functools.partial(
    pl.kernel,
    out_type=jax.ShapeDtypeStruct((F_IN * NC * (NS // F_IN) * NP,),
                                  jnp.float32),
    mesh=_mesh,
    compiler_params=pltpu.CompilerParams(needs_layout_passes=False),
    scratch_types=[
        pltpu.VMEM((NP,), jnp.float32),
        pltpu.VMEM((NP,), jnp.float32),
        pltpu.VMEM((2 * CHA,), jnp.int32),
        pltpu.VMEM((2 * CHA,), jnp.int32),
        pltpu.VMEM((2 * CHA,), jnp.float32),
        pltpu.SemaphoreType.DMA,
        pltpu.SemaphoreType.DMA,
    ],
)
def _agg_sc(src_hbm, dst_hbm, w_hbm, x0, x1, x2, x3, dis_hbm, out_hbm,
            col, acc, si, di, wv, sem0, sem1):
    c = lax.axis_index("c")
    s = lax.axis_index("s")
    f = s % F_IN
    g = s // F_IN
    xf = (x0, x1, x2, x3)
    sems = (sem0, sem1)

    pltpu.sync_copy(dis_hbm.at[pl.ds(0, NP)], acc)
    for ff in range(F_IN):

        @pl.when(f == ff)
        def _():
            pltpu.sync_copy(xf[ff].at[pl.ds(0, NP)], col)

    @plsc.parallel_loop(0, NP // 16, unroll=8)
    def _(i):
        d16 = pl.ds(pl.multiple_of(i * 16, 16), 16)
        col[d16] = col[d16] * acc[d16]
    _zero_vmem(acc, NP)

    base = c * (EP // NC) + g * GT
    nk = GT // CHA

    def _load(k, b):
        off = base + k * CHA
        bs = pl.ds(b * CHA, CHA)
        return (
            pltpu.async_copy(src_hbm.at[pl.ds(off, CHA)], si.at[bs], sems[b]),
            pltpu.async_copy(dst_hbm.at[pl.ds(off, CHA)], di.at[bs], sems[b]),
            pltpu.async_copy(w_hbm.at[pl.ds(off, CHA)], wv.at[bs], sems[b]),
        )

    descs = _load(0, 0)
    for k in range(nk):
        b = k % 2
        for d in descs:
            d.wait()
        if k + 1 < nk:
            descs = _load(k + 1, (k + 1) % 2)

        bo = b * CHA

        def body(i, carry):
            for u in range(4):
                d16 = pl.ds(pl.multiple_of(bo + i * 64 + u * 16, 16), 16)
                v = plsc.load_gather(col, [si[d16]])
                plsc.addupdate_scatter(acc, [di[d16]], v * wv[d16])
            return carry

        lax.fori_loop(0, CHA // 64, body, 0)
    slot = f * (NC * (NS // F_IN)) + c * (NS // F_IN) + g
    pltpu.sync_copy(acc, out_hbm.at[pl.ds(slot * NP, NP)])


@functools.partial(
    pl.kernel,
    out_type=jax.ShapeDtypeStruct((NW * NP,), jnp.float32),
    mesh=_mesh,
    compiler_params=pltpu.CompilerParams(needs_layout_passes=False),
    scratch_types=[
        pltpu.VMEM((NP,), jnp.float32),
        pltpu.VMEM((NP,), jnp.float32),
        pltpu.VMEM((CH,), jnp.int32),
        pltpu.VMEM((CH,), jnp.int32),
        pltpu.VMEM((CH,), jnp.float32),
    ],
)
def _out_sc(src_hbm, dst_hbm, w_hbm, ys_hbm, out_hbm, col, acc, si, di, wv):
    c = lax.axis_index("c")
    s = lax.axis_index("s")
    pltpu.sync_copy(ys_hbm.at[pl.ds(0, NP)], col)
    _zero_vmem(acc, NP)
    base = c * (EP // NC) + s * PT
    for k in range(PT // CH):
        off = base + k * CH
        pltpu.sync_copy(src_hbm.at[pl.ds(off, CH)], si)
        pltpu.sync_copy(dst_hbm.at[pl.ds(off, CH)], di)
        pltpu.sync_copy(w_hbm.at[pl.ds(off, CH)], wv)

        def body(i, carry):
            for u in range(4):
                d16 = pl.ds(pl.multiple_of(i * 64 + u * 16, 16), 16)
                v = plsc.load_gather(col, [si[d16]])
                plsc.addupdate_scatter(acc, [di[d16]], v * wv[d16])
            return carry

        lax.fori_loop(0, CH // 64, body, 0)
    pltpu.sync_copy(acc, out_hbm.at[pl.ds((c * NS + s) * NP, NP)])


def _p2_body(degp_ref, dis_ref, selfn_ref):
    deg = jnp.sum(degp_ref[...], axis=0, keepdims=True) + 1.0
    dis = lax.rsqrt(deg)
    dis_ref[...] = dis
    selfn_ref[...] = dis * dis


_p2_tc = pl.pallas_call(
    _p2_body,
    out_shape=(
        jax.ShapeDtypeStruct((1, NP), jnp.float32),
        jax.ShapeDtypeStruct((1, NP), jnp.float32),
    ),
)


_B4 = NP // 8
_G3 = NC * (NS // F_IN)


def _p4_body(ag0, ag1, ag2, ag3, xt_ref, dis_ref, selfn_ref,
             w1t_ref, b1c_ref, w2_ref, b2s_ref, ys_ref, oinit_ref):
    dis = dis_ref[...]
    selfn = selfn_ref[...]
    xt = xt_ref[...]
    aggs = []
    for f, ag in enumerate((ag0, ag1, ag2, ag3)):
        tot = jnp.sum(ag[...], axis=0, keepdims=True)
        aggs.append(dis * tot + selfn * xt[f:f + 1, :])
    w1t = w1t_ref[...]
    h = b1c_ref[...]
    for f in range(F_IN):
        h = h + w1t[:, f:f + 1] * aggs[f]
    h = jnp.maximum(h, 0.0)
    y = jnp.sum(h * w2_ref[...], axis=0, keepdims=True)
    ys_ref[...] = y * dis
    oinit_ref[...] = y * selfn + b2s_ref[...]


_p4_tc = pl.pallas_call(
    _p4_body,
    grid=(NP // _B4,),
    in_specs=[
        pl.BlockSpec((_G3, _B4), lambda i: (0, i)),
        pl.BlockSpec((_G3, _B4), lambda i: (0, i)),
        pl.BlockSpec((_G3, _B4), lambda i: (0, i)),
        pl.BlockSpec((_G3, _B4), lambda i: (0, i)),
        pl.BlockSpec((F_IN, _B4), lambda i: (0, i)),
        pl.BlockSpec((1, _B4), lambda i: (0, i)),
        pl.BlockSpec((1, _B4), lambda i: (0, i)),
        pl.BlockSpec((H, F_IN), lambda i: (0, 0)),
        pl.BlockSpec((H, 1), lambda i: (0, 0)),
        pl.BlockSpec((H, 1), lambda i: (0, 0)),
        pl.BlockSpec((1, 1), lambda i: (0, 0)),
    ],
    out_specs=(
        pl.BlockSpec((1, _B4), lambda i: (0, i)),
        pl.BlockSpec((1, _B4), lambda i: (0, i)),
    ),
    out_shape=(
        jax.ShapeDtypeStruct((1, NP), jnp.float32),
        jax.ShapeDtypeStruct((1, NP), jnp.float32),
    ),
)


def _p6_body(outp_ref, dis_ref, oinit_ref, out_ref):
    tot = jnp.sum(outp_ref[...], axis=0, keepdims=True)
    out_ref[...] = oinit_ref[...] + dis_ref[...] * tot


_p6_tc = pl.pallas_call(
    _p6_body,
    out_shape=jax.ShapeDtypeStruct((1, NP), jnp.float32),
)


def kernel(x, edge_index, edge_attr, W1, b1, W2, b2):
    f32 = jnp.float32
    pad_e = EP - E
    srcp = jnp.concatenate(
        [edge_index[0], jnp.full((pad_e,), N, dtype=jnp.int32)])
    dstp = jnp.concatenate(
        [edge_index[1], jnp.full((pad_e,), N, dtype=jnp.int32)])
    wp = jnp.concatenate([edge_attr.astype(f32), jnp.zeros((pad_e,), f32)])
    xpad = jnp.pad(x.astype(f32), ((0, NP - N), (0, 0)))
    xt = xpad.T
    xcols = [xt[ff].reshape(NP) for ff in range(F_IN)]

    degp = _deg_sc(dstp, wp).reshape(NW, NP)
    dis, selfn = _p2_tc(degp)
    agg = _agg_sc(srcp, dstp, wp, xcols[0], xcols[1], xcols[2], xcols[3],
                  dis.reshape(NP)).reshape(F_IN, _G3, NP)
    ys, oinit = _p4_tc(agg[0], agg[1], agg[2], agg[3], xt, dis, selfn,
                       W1.astype(f32).T, b1.astype(f32).reshape(H, 1),
                       W2.astype(f32), b2.astype(f32).reshape(1, 1))
    outp = _out_sc(srcp, dstp, wp, ys.reshape(NP)).reshape(NW, NP)
    out = _p6_tc(outp, dis, oinit)
    return out[0, :N].reshape(N, 1)


# speedup vs baseline: 1.1762x; 1.0468x over previous
import functools

import jax
import jax.numpy as jnp
from jax import lax
from jax.experimental import pallas as pl
from jax.experimental.pallas import tpu as pltpu
from jax.experimental.pallas import tpu_sc as plsc

N = 50000
E = 800000
F_IN = 4
H = 64

NP = 50176
EP = 819200
NC = 2
NS = 16
NW = NC * NS
PT = EP // NW
GT = PT * F_IN
CH = 6400
CHA = 3200
ZS = NP // NS

_mesh = plsc.VectorSubcoreMesh(core_axis_name="c", subcore_axis_name="s")


def _zero_vmem(buf, n):
    z = jnp.zeros((16,), jnp.float32)

    @plsc.parallel_loop(0, n // 16, unroll=8)
    def _(i):
        buf[pl.ds(pl.multiple_of(i * 16, 16), 16)] = z


@functools.partial(
    pl.kernel,
    out_type=jax.ShapeDtypeStruct((NW * NP,), jnp.float32),
    mesh=_mesh,
    compiler_params=pltpu.CompilerParams(needs_layout_passes=False),
    scratch_types=[
        pltpu.VMEM((NP,), jnp.float32),
        pltpu.VMEM((2 * CH,), jnp.int32),
        pltpu.VMEM((2 * CH,), jnp.float32),
        pltpu.SemaphoreType.DMA,
        pltpu.SemaphoreType.DMA,
    ],
)
def _deg_sc(dst_hbm, w_hbm, out_hbm, acc, di, wv, sem0, sem1):
    c = lax.axis_index("c")
    s = lax.axis_index("s")
    sems = (sem0, sem1)
    _zero_vmem(acc, NP)
    base = c * (EP // NC) + s * PT
    nk = PT // CH

    def _load(k, b):
        off = base + k * CH
        bs = pl.ds(b * CH, CH)
        return (
            pltpu.async_copy(dst_hbm.at[pl.ds(off, CH)], di.at[bs], sems[b]),
            pltpu.async_copy(w_hbm.at[pl.ds(off, CH)], wv.at[bs], sems[b]),
        )

    descs = _load(0, 0)
    for k in range(nk):
        b = k % 2
        for d in descs:
            d.wait()
        if k + 1 < nk:
            descs = _load(k + 1, (k + 1) % 2)

        bo = b * CH

        def body(i, carry):
            for u in range(4):
                d16 = pl.ds(pl.multiple_of(bo + i * 64 + u * 16, 16), 16)
                plsc.addupdate_scatter(acc, [di[d16]], wv[d16])
            return carry

        lax.fori_loop(0, CH // 64, body, 0)
    pltpu.sync_copy(acc, out_hbm.at[pl.ds((c * NS + s) * NP, NP)])


@functools.partial(
    pl.kernel,
    out_type=jax.ShapeDtypeStruct((F_IN * NC * (NS // F_IN) * NP,),
                                  jnp.float32),
    mesh=_mesh,
    compiler_params=pltpu.CompilerParams(needs_layout_passes=False),
    scratch_types=[
        pltpu.VMEM((NP,), jnp.float32),
        pltpu.VMEM((NP,), jnp.float32),
        pltpu.VMEM((2 * CHA,), jnp.int32),
        pltpu.VMEM((2 * CHA,), jnp.int32),
        pltpu.VMEM((2 * CHA,), jnp.float32),
        pltpu.SemaphoreType.DMA,
        pltpu.SemaphoreType.DMA,
    ],
)
def _agg_sc(src_hbm, dst_hbm, w_hbm, x0, x1, x2, x3, dis_hbm, out_hbm,
            col, acc, si, di, wv, sem0, sem1):
    c = lax.axis_index("c")
    s = lax.axis_index("s")
    f = s % F_IN
    g = s // F_IN
    xf = (x0, x1, x2, x3)
    sems = (sem0, sem1)

    pltpu.sync_copy(dis_hbm.at[pl.ds(0, NP)], acc)
    for ff in range(F_IN):

        @pl.when(f == ff)
        def _():
            pltpu.sync_copy(xf[ff].at[pl.ds(0, NP)], col)

    @plsc.parallel_loop(0, NP // 16, unroll=8)
    def _(i):
        d16 = pl.ds(pl.multiple_of(i * 16, 16), 16)
        col[d16] = col[d16] * acc[d16]
    _zero_vmem(acc, NP)

    base = c * (EP // NC) + g * GT
    nk = GT // CHA

    def _load(k, b):
        off = base + k * CHA
        bs = pl.ds(b * CHA, CHA)
        return (
            pltpu.async_copy(src_hbm.at[pl.ds(off, CHA)], si.at[bs], sems[b]),
            pltpu.async_copy(dst_hbm.at[pl.ds(off, CHA)], di.at[bs], sems[b]),
            pltpu.async_copy(w_hbm.at[pl.ds(off, CHA)], wv.at[bs], sems[b]),
        )

    descs = _load(0, 0)
    for k in range(nk):
        b = k % 2
        for d in descs:
            d.wait()
        if k + 1 < nk:
            descs = _load(k + 1, (k + 1) % 2)

        bo = b * CHA

        def body(i, carry):
            for u in range(4):
                d16 = pl.ds(pl.multiple_of(bo + i * 64 + u * 16, 16), 16)
                v = plsc.load_gather(col, [si[d16]])
                plsc.addupdate_scatter(acc, [di[d16]], v * wv[d16])
            return carry

        lax.fori_loop(0, CHA // 64, body, 0)
    slot = f * (NC * (NS // F_IN)) + c * (NS // F_IN) + g
    pltpu.sync_copy(acc, out_hbm.at[pl.ds(slot * NP, NP)])


@functools.partial(
    pl.kernel,
    out_type=jax.ShapeDtypeStruct((NW * NP,), jnp.float32),
    mesh=_mesh,
    compiler_params=pltpu.CompilerParams(needs_layout_passes=False),
    scratch_types=[
        pltpu.VMEM((NP,), jnp.float32),
        pltpu.VMEM((NP,), jnp.float32),
        pltpu.VMEM((2 * CHA,), jnp.int32),
        pltpu.VMEM((2 * CHA,), jnp.int32),
        pltpu.VMEM((2 * CHA,), jnp.float32),
        pltpu.SemaphoreType.DMA,
        pltpu.SemaphoreType.DMA,
    ],
)
def _out_sc(src_hbm, dst_hbm, w_hbm, ys_hbm, out_hbm, col, acc, si, di, wv,
            sem0, sem1):
    c = lax.axis_index("c")
    s = lax.axis_index("s")
    sems = (sem0, sem1)
    pltpu.sync_copy(ys_hbm.at[pl.ds(0, NP)], col)
    _zero_vmem(acc, NP)
    base = c * (EP // NC) + s * PT
    nk = PT // CHA

    def _load(k, b):
        off = base + k * CHA
        bs = pl.ds(b * CHA, CHA)
        return (
            pltpu.async_copy(src_hbm.at[pl.ds(off, CHA)], si.at[bs], sems[b]),
            pltpu.async_copy(dst_hbm.at[pl.ds(off, CHA)], di.at[bs], sems[b]),
            pltpu.async_copy(w_hbm.at[pl.ds(off, CHA)], wv.at[bs], sems[b]),
        )

    descs = _load(0, 0)
    for k in range(nk):
        b = k % 2
        for d in descs:
            d.wait()
        if k + 1 < nk:
            descs = _load(k + 1, (k + 1) % 2)

        bo = b * CHA

        def body(i, carry):
            for u in range(4):
                d16 = pl.ds(pl.multiple_of(bo + i * 64 + u * 16, 16), 16)
                v = plsc.load_gather(col, [si[d16]])
                plsc.addupdate_scatter(acc, [di[d16]], v * wv[d16])
            return carry

        lax.fori_loop(0, CHA // 64, body, 0)
    pltpu.sync_copy(acc, out_hbm.at[pl.ds((c * NS + s) * NP, NP)])


def _p2_body(degp_ref, dis_ref, selfn_ref):
    deg = jnp.sum(degp_ref[...], axis=0, keepdims=True) + 1.0
    dis = lax.rsqrt(deg)
    dis_ref[...] = dis
    selfn_ref[...] = dis * dis


_p2_tc = pl.pallas_call(
    _p2_body,
    out_shape=(
        jax.ShapeDtypeStruct((1, NP), jnp.float32),
        jax.ShapeDtypeStruct((1, NP), jnp.float32),
    ),
)


_B4 = NP // 8
_G3 = NC * (NS // F_IN)


def _p4_body(ag0, ag1, ag2, ag3, xt_ref, dis_ref, selfn_ref,
             w1t_ref, b1c_ref, w2_ref, b2s_ref, ys_ref, oinit_ref):
    dis = dis_ref[...]
    selfn = selfn_ref[...]
    xt = xt_ref[...]
    aggs = []
    for f, ag in enumerate((ag0, ag1, ag2, ag3)):
        tot = jnp.sum(ag[...], axis=0, keepdims=True)
        aggs.append(dis * tot + selfn * xt[f:f + 1, :])
    w1t = w1t_ref[...]
    h = b1c_ref[...]
    for f in range(F_IN):
        h = h + w1t[:, f:f + 1] * aggs[f]
    h = jnp.maximum(h, 0.0)
    y = jnp.sum(h * w2_ref[...], axis=0, keepdims=True)
    ys_ref[...] = y * dis
    oinit_ref[...] = y * selfn + b2s_ref[...]


_p4_tc = pl.pallas_call(
    _p4_body,
    grid=(NP // _B4,),
    in_specs=[
        pl.BlockSpec((_G3, _B4), lambda i: (0, i)),
        pl.BlockSpec((_G3, _B4), lambda i: (0, i)),
        pl.BlockSpec((_G3, _B4), lambda i: (0, i)),
        pl.BlockSpec((_G3, _B4), lambda i: (0, i)),
        pl.BlockSpec((F_IN, _B4), lambda i: (0, i)),
        pl.BlockSpec((1, _B4), lambda i: (0, i)),
        pl.BlockSpec((1, _B4), lambda i: (0, i)),
        pl.BlockSpec((H, F_IN), lambda i: (0, 0)),
        pl.BlockSpec((H, 1), lambda i: (0, 0)),
        pl.BlockSpec((H, 1), lambda i: (0, 0)),
        pl.BlockSpec((1, 1), lambda i: (0, 0)),
    ],
    out_specs=(
        pl.BlockSpec((1, _B4), lambda i: (0, i)),
        pl.BlockSpec((1, _B4), lambda i: (0, i)),
    ),
    out_shape=(
        jax.ShapeDtypeStruct((1, NP), jnp.float32),
        jax.ShapeDtypeStruct((1, NP), jnp.float32),
    ),
)


def _p6_body(outp_ref, dis_ref, oinit_ref, out_ref):
    tot = jnp.sum(outp_ref[...], axis=0, keepdims=True)
    out_ref[...] = oinit_ref[...] + dis_ref[...] * tot


_p6_tc = pl.pallas_call(
    _p6_body,
    out_shape=jax.ShapeDtypeStruct((1, NP), jnp.float32),
)


def kernel(x, edge_index, edge_attr, W1, b1, W2, b2):
    f32 = jnp.float32
    pad_e = EP - E
    srcp = jnp.concatenate(
        [edge_index[0], jnp.full((pad_e,), N, dtype=jnp.int32)])
    dstp = jnp.concatenate(
        [edge_index[1], jnp.full((pad_e,), N, dtype=jnp.int32)])
    wp = jnp.concatenate([edge_attr.astype(f32), jnp.zeros((pad_e,), f32)])
    xpad = jnp.pad(x.astype(f32), ((0, NP - N), (0, 0)))
    xt = xpad.T
    xcols = [xt[ff].reshape(NP) for ff in range(F_IN)]

    degp = _deg_sc(dstp, wp).reshape(NW, NP)
    dis, selfn = _p2_tc(degp)
    agg = _agg_sc(srcp, dstp, wp, xcols[0], xcols[1], xcols[2], xcols[3],
                  dis.reshape(NP)).reshape(F_IN, _G3, NP)
    ys, oinit = _p4_tc(agg[0], agg[1], agg[2], agg[3], xt, dis, selfn,
                       W1.astype(f32).T, b1.astype(f32).reshape(H, 1),
                       W2.astype(f32), b2.astype(f32).reshape(1, 1))
    outp = _out_sc(srcp, dstp, wp, ys.reshape(NP)).reshape(NW, NP)
    out = _p6_tc(outp, dis, oinit)
    return out[0, :N].reshape(N, 1)
